# baseline (speedup 1.0000x reference)
import functools

import jax
import jax.numpy as jnp
from jax.experimental import pallas as pl
from jax.experimental.pallas import tpu as pltpu

NUM_ITEMS = 100000
HIDDEN = 64
BATCH = 1024
SEQ = 50
TOP_K = 10

BJ = 2048


def _logits_body(fused_ref, w_ref, b_ref, out_ref):
    acc = jnp.dot(fused_ref[...], w_ref[...], preferred_element_type=jnp.float32)
    out_ref[...] = acc + b_ref[...]


def _logits_matmul(fused, W_proj, b_proj2d):
    grid = (pl.cdiv(NUM_ITEMS, BJ),)
    return pl.pallas_call(
        _logits_body,
        grid=grid,
        in_specs=[
            pl.BlockSpec((BATCH, HIDDEN), lambda j: (0, 0)),
            pl.BlockSpec((HIDDEN, BJ), lambda j: (0, j)),
            pl.BlockSpec((1, BJ), lambda j: (0, j)),
        ],
        out_specs=pl.BlockSpec((BATCH, BJ), lambda j: (0, j)),
        out_shape=jax.ShapeDtypeStruct((BATCH, NUM_ITEMS), jnp.float32),
        compiler_params=pltpu.CompilerParams(
            dimension_semantics=("arbitrary",),
        ),
    )(fused, W_proj, b_proj2d)


def kernel(sequence_ids, item_embeddings, W_rec, b_rec, W_gate, b_gate, W_proj, b_proj):
    mask = (sequence_ids > 0).astype(jnp.float32)[..., None]
    safe_idx = jnp.maximum(sequence_ids - 1, 0)
    seq_emb = jnp.take(item_embeddings, safe_idx, axis=0) * mask
    pooled = seq_emb.sum(axis=1) / jnp.maximum(mask.sum(axis=1), 1.0)
    user_rep = jnp.tanh(pooled @ W_rec + b_rec)
    scores = user_rep @ item_embeddings.T
    _, indices = jax.lax.top_k(scores, TOP_K)
    retrieved = jnp.take(item_embeddings, indices, axis=0).mean(axis=1)
    concat = jnp.concatenate([user_rep, retrieved], axis=1)
    gate = jax.nn.sigmoid(concat @ W_gate + b_gate)
    fused = gate * user_rep + (1.0 - gate) * retrieved
    return _logits_matmul(fused, W_proj, b_proj.reshape(1, NUM_ITEMS))


# speedup vs baseline: 2.9632x; 2.9632x over previous
import jax
import jax.numpy as jnp
from jax.experimental import pallas as pl
from jax.experimental.pallas import tpu as pltpu

NUM_ITEMS = 100000
HIDDEN = 64
BATCH = 1024
SEQ = 50
TOP_K = 10

BJ = 2048
G = 16
NBLK = 49
NG_PAD = NBLK * BJ // G
NCAND = TOP_K * G
NEG = -3.0e38
PAD_VAL = -1.0e30
BIG_I = 2 ** 30


def _pool_body(rows_ref, ids_ref, w_ref, b_ref, out_ref):
    ids = ids_ref[...]
    mask = (ids > 0).astype(jnp.float32)
    rows = rows_ref[...] * mask[:, :, None]
    pooled = jnp.sum(rows, axis=1)
    cnt = jnp.maximum(jnp.sum(mask, axis=1, keepdims=True), 1.0)
    pooled = pooled / cnt
    out_ref[...] = jnp.tanh(
        jnp.dot(pooled, w_ref[...], preferred_element_type=jnp.float32)
        + b_ref[...])


def _groupmax_body(emb_ref, ut_ref, mt_ref):
    st = jnp.dot(emb_ref[...], ut_ref[...],
                 preferred_element_type=jnp.float32)
    j = pl.program_id(0)
    row = jax.lax.broadcasted_iota(jnp.int32, (BJ, BATCH), 0) + j * BJ
    st = jnp.where(row < NUM_ITEMS, st, PAD_VAL)
    mt_ref[...] = jnp.max(st.reshape(BJ // G, G, BATCH), axis=1)


def _topgrp_body(m_ref, out_ref):
    mv = m_ref[...]
    cols = jax.lax.broadcasted_iota(jnp.int32, mv.shape, 1)
    for k in range(TOP_K):
        m = jnp.max(mv, axis=1, keepdims=True)
        idx = jnp.min(jnp.where(mv == m, cols, BIG_I), axis=1, keepdims=True)
        out_ref[:, pl.ds(k, 1)] = idx
        mv = jnp.where(cols == idx, NEG, mv)
    out_ref[:, pl.ds(TOP_K, 6)] = jnp.zeros((mv.shape[0], 6), jnp.int32)


def _rescore_body(e_ref, u_ref, ids_ref, out_ref):
    e = e_ref[...]
    u = u_ref[...]
    ids = ids_ref[...]
    s = jnp.sum(e * u[:, None, :], axis=2)
    for k in range(TOP_K):
        m = jnp.max(s, axis=1, keepdims=True)
        itm = jnp.min(jnp.where(s == m, ids, BIG_I), axis=1, keepdims=True)
        out_ref[:, pl.ds(k, 1)] = itm
        s = jnp.where(ids == itm, NEG, s)
    out_ref[:, pl.ds(TOP_K, 6)] = jnp.zeros((s.shape[0], 6), jnp.int32)


def _fuse_body(rows_ref, u_ref, wg_ref, bg_ref, out_ref):
    ret = jnp.mean(rows_ref[...], axis=1)
    u = u_ref[...]
    wg = wg_ref[...]
    gate = jax.nn.sigmoid(
        jnp.dot(u, wg[:HIDDEN], preferred_element_type=jnp.float32)
        + jnp.dot(ret, wg[HIDDEN:], preferred_element_type=jnp.float32)
        + bg_ref[...])
    out_ref[...] = gate * u + (1.0 - gate) * ret


def _logits_body(fused_ref, w_ref, b_ref, out_ref):
    out_ref[...] = jnp.dot(fused_ref[...], w_ref[...],
                           preferred_element_type=jnp.float32) + b_ref[...]


def kernel(sequence_ids, item_embeddings, W_rec, b_rec, W_gate, b_gate, W_proj, b_proj):
    sequence_ids = sequence_ids.astype(jnp.int32)
    safe_idx = jnp.maximum(sequence_ids - 1, 0)
    seq_rows = jnp.take(item_embeddings, safe_idx.reshape(-1), axis=0)
    seq_rows = seq_rows.reshape(BATCH, SEQ, HIDDEN)

    user_rep = pl.pallas_call(
        _pool_body,
        grid=(4,),
        in_specs=[
            pl.BlockSpec((BATCH // 4, SEQ, HIDDEN), lambda i: (i, 0, 0)),
            pl.BlockSpec((BATCH // 4, SEQ), lambda i: (i, 0)),
            pl.BlockSpec((HIDDEN, HIDDEN), lambda i: (0, 0)),
            pl.BlockSpec((1, HIDDEN), lambda i: (0, 0)),
        ],
        out_specs=pl.BlockSpec((BATCH // 4, HIDDEN), lambda i: (i, 0)),
        out_shape=jax.ShapeDtypeStruct((BATCH, HIDDEN), jnp.float32),
    )(seq_rows, sequence_ids, W_rec, b_rec.reshape(1, HIDDEN))

    m_t = pl.pallas_call(
        _groupmax_body,
        grid=(NBLK,),
        in_specs=[
            pl.BlockSpec((BJ, HIDDEN), lambda j: (j, 0)),
            pl.BlockSpec((HIDDEN, BATCH), lambda j: (0, 0)),
        ],
        out_specs=pl.BlockSpec((BJ // G, BATCH), lambda j: (j, 0)),
        out_shape=jax.ShapeDtypeStruct((NG_PAD, BATCH), jnp.float32),
        compiler_params=pltpu.CompilerParams(
            dimension_semantics=("arbitrary",)),
    )(item_embeddings, user_rep.T)

    m = m_t.T

    grp_idx = pl.pallas_call(
        _topgrp_body,
        grid=(4,),
        in_specs=[pl.BlockSpec((BATCH // 4, NG_PAD), lambda i: (i, 0))],
        out_specs=pl.BlockSpec((BATCH // 4, 16), lambda i: (i, 0)),
        out_shape=jax.ShapeDtypeStruct((BATCH, 16), jnp.int32),
    )(m)

    cand_ids = (grp_idx[:, :TOP_K, None] * G
                + jnp.arange(G, dtype=jnp.int32)[None, None, :])
    cand_ids = jnp.minimum(cand_ids.reshape(BATCH, NCAND), NUM_ITEMS - 1)
    e_cand = jnp.take(item_embeddings, cand_ids.reshape(-1), axis=0)
    e_cand = e_cand.reshape(BATCH, NCAND, HIDDEN)

    top_ids = pl.pallas_call(
        _rescore_body,
        grid=(8,),
        in_specs=[
            pl.BlockSpec((BATCH // 8, NCAND, HIDDEN), lambda i: (i, 0, 0)),
            pl.BlockSpec((BATCH // 8, HIDDEN), lambda i: (i, 0)),
            pl.BlockSpec((BATCH // 8, NCAND), lambda i: (i, 0)),
        ],
        out_specs=pl.BlockSpec((BATCH // 8, 16), lambda i: (i, 0)),
        out_shape=jax.ShapeDtypeStruct((BATCH, 16), jnp.int32),
    )(e_cand, user_rep, cand_ids)

    ret_rows = jnp.take(item_embeddings, top_ids[:, :TOP_K].reshape(-1), axis=0)
    ret_rows = ret_rows.reshape(BATCH, TOP_K, HIDDEN)

    fused = pl.pallas_call(
        _fuse_body,
        grid=(4,),
        in_specs=[
            pl.BlockSpec((BATCH // 4, TOP_K, HIDDEN), lambda i: (i, 0, 0)),
            pl.BlockSpec((BATCH // 4, HIDDEN), lambda i: (i, 0)),
            pl.BlockSpec((2 * HIDDEN, HIDDEN), lambda i: (0, 0)),
            pl.BlockSpec((1, HIDDEN), lambda i: (0, 0)),
        ],
        out_specs=pl.BlockSpec((BATCH // 4, HIDDEN), lambda i: (i, 0)),
        out_shape=jax.ShapeDtypeStruct((BATCH, HIDDEN), jnp.float32),
    )(ret_rows, user_rep, W_gate, b_gate.reshape(1, HIDDEN))

    return pl.pallas_call(
        _logits_body,
        grid=(NBLK,),
        in_specs=[
            pl.BlockSpec((BATCH, HIDDEN), lambda j: (0, 0)),
            pl.BlockSpec((HIDDEN, BJ), lambda j: (0, j)),
            pl.BlockSpec((1, BJ), lambda j: (0, j)),
        ],
        out_specs=pl.BlockSpec((BATCH, BJ), lambda j: (0, j)),
        out_shape=jax.ShapeDtypeStruct((BATCH, NUM_ITEMS), jnp.float32),
        compiler_params=pltpu.CompilerParams(
            dimension_semantics=("arbitrary",)),
    )(fused, W_proj, b_proj.reshape(1, NUM_ITEMS))
